# Initial kernel scaffold; baseline (speedup 1.0000x reference)
import jax
import jax.numpy as jnp
from jax.experimental import pallas as pl


def kernel(commands, args, groups, command_embed, arg_embed, W_fcn, b_fcn, group_embed, pos_embed):
    raise NotImplementedError("write your pallas kernel here")


# speedup vs baseline: 11.4239x; 11.4239x over previous
import jax
import jax.numpy as jnp
from jax.experimental import pallas as pl

S, GN = 512, 256
N_COMMANDS = 7
ARGS_DIM = 256
N_ARGS = 11
D_MODEL = 256
ARG_EMB_ROWS = ARGS_DIM + 1
GROUP_ROWS = 10

T = S * GN
SB = 8
NT = SB * GN


def _body(cmd_ref, args_ref, grp_ref, ce_ref, ae_ref, w_ref, b_ref, ge_ref, pe_ref, out_ref):
    f32 = jnp.float32
    cmd = cmd_ref[...]
    grp = grp_ref[...]
    args = args_ref[...]

    ioc = jax.lax.broadcasted_iota(jnp.int32, (NT, N_COMMANDS), 1)
    acc = jnp.dot((ioc == cmd).astype(f32), ce_ref[...],
                  preferred_element_type=f32)
    iog = jax.lax.broadcasted_iota(jnp.int32, (NT, GROUP_ROWS), 1)
    acc = acc + jnp.dot((iog == grp).astype(f32), ge_ref[...],
                        preferred_element_type=f32)

    tok_row = jax.lax.broadcasted_iota(jnp.int32, (NT, SB), 0) // GN
    rep = (tok_row == jax.lax.broadcasted_iota(jnp.int32, (NT, SB), 1)).astype(f32)
    acc = acc + jnp.dot(rep, pe_ref[...], preferred_element_type=f32)

    ioa = jax.lax.broadcasted_iota(jnp.int32, (NT, ARG_EMB_ROWS), 1)
    for j in range(N_ARGS):
        aj = args[:, j:j + 1] + 1
        oh = (ioa == aj).astype(f32)
        a_j = jnp.dot(oh, ae_ref[...], preferred_element_type=f32)
        w_j = w_ref[:, j * 64:(j + 1) * 64]
        acc = acc + jax.lax.dot_general(
            a_j, w_j, (((1,), (1,)), ((), ())),
            preferred_element_type=f32)

    out_ref[...] = acc + b_ref[...]


def kernel(commands, args, groups, command_embed, arg_embed, W_fcn, b_fcn, group_embed, pos_embed):
    cmdf = commands.astype(jnp.int32).reshape(T, 1)
    argsf = args.astype(jnp.int32).reshape(T, N_ARGS)
    grpf = groups.astype(jnp.int32).reshape(T, 1)
    bf = b_fcn.reshape(1, D_MODEL)

    grid = (S // SB,)
    out = pl.pallas_call(
        _body,
        grid=grid,
        in_specs=[
            pl.BlockSpec((NT, 1), lambda i: (i, 0)),
            pl.BlockSpec((NT, N_ARGS), lambda i: (i, 0)),
            pl.BlockSpec((NT, 1), lambda i: (i, 0)),
            pl.BlockSpec((N_COMMANDS, D_MODEL), lambda i: (0, 0)),
            pl.BlockSpec((ARG_EMB_ROWS, 64), lambda i: (0, 0)),
            pl.BlockSpec((D_MODEL, 64 * N_ARGS), lambda i: (0, 0)),
            pl.BlockSpec((1, D_MODEL), lambda i: (0, 0)),
            pl.BlockSpec((GROUP_ROWS, D_MODEL), lambda i: (0, 0)),
            pl.BlockSpec((SB, D_MODEL), lambda i: (i, 0)),
        ],
        out_specs=pl.BlockSpec((NT, D_MODEL), lambda i: (i, 0)),
        out_shape=jax.ShapeDtypeStruct((T, D_MODEL), jnp.float32),
    )(cmdf, argsf, grpf, command_embed, arg_embed, W_fcn, bf,
      group_embed, pos_embed)
    return out.reshape(S, GN, D_MODEL)


# speedup vs baseline: 14.0766x; 1.2322x over previous
import jax
import jax.numpy as jnp
from jax.experimental import pallas as pl

S, GN = 512, 256
N_COMMANDS = 7
ARGS_DIM = 256
N_ARGS = 11
D_MODEL = 256
ARG_EMB_ROWS = ARGS_DIM + 1
GROUP_ROWS = 10

T = S * GN
SB = 8
NT = SB * GN


def _body(cmd_ref, args_ref, grp_ref, ce_ref, ae_ref, w_ref, b_ref, ge_ref, pe_ref, out_ref):
    f32 = jnp.float32
    bf16 = jnp.bfloat16
    cmd = cmd_ref[...]
    grp = grp_ref[...]
    args = args_ref[...]

    ioc = jax.lax.broadcasted_iota(jnp.int32, (NT, N_COMMANDS), 1)
    acc = jnp.dot((ioc == cmd).astype(bf16), ce_ref[...],
                  preferred_element_type=f32)
    iog = jax.lax.broadcasted_iota(jnp.int32, (NT, GROUP_ROWS), 1)
    acc = acc + jnp.dot((iog == grp).astype(bf16), ge_ref[...],
                        preferred_element_type=f32)

    tok_row = jax.lax.broadcasted_iota(jnp.int32, (NT, SB), 0) // GN
    rep = (tok_row == jax.lax.broadcasted_iota(jnp.int32, (NT, SB), 1)).astype(bf16)
    acc = acc + jnp.dot(rep, pe_ref[...], preferred_element_type=f32)

    ioa = jax.lax.broadcasted_iota(jnp.int32, (NT, ARG_EMB_ROWS), 1)
    a_parts = []
    for j in range(N_ARGS):
        aj = args[:, j:j + 1] + 1
        oh = (ioa == aj).astype(bf16)
        a_parts.append(jnp.dot(oh, ae_ref[...],
                               preferred_element_type=f32))
    a = jnp.concatenate(a_parts, axis=1).astype(bf16)
    acc = acc + jax.lax.dot_general(
        a, w_ref[...], (((1,), (1,)), ((), ())),
        preferred_element_type=f32)

    out_ref[...] = acc + b_ref[...]


def kernel(commands, args, groups, command_embed, arg_embed, W_fcn, b_fcn, group_embed, pos_embed):
    cmdf = commands.astype(jnp.int32).reshape(T, 1)
    argsf = args.astype(jnp.int32).reshape(T, N_ARGS)
    grpf = groups.astype(jnp.int32).reshape(T, 1)
    bf = b_fcn.reshape(1, D_MODEL)

    ce = command_embed.astype(jnp.bfloat16)
    ae = arg_embed.astype(jnp.bfloat16)
    w = W_fcn.astype(jnp.bfloat16)
    ge = group_embed.astype(jnp.bfloat16)
    pe = pos_embed.astype(jnp.bfloat16)

    grid = (S // SB,)
    out = pl.pallas_call(
        _body,
        grid=grid,
        in_specs=[
            pl.BlockSpec((NT, 1), lambda i: (i, 0)),
            pl.BlockSpec((NT, N_ARGS), lambda i: (i, 0)),
            pl.BlockSpec((NT, 1), lambda i: (i, 0)),
            pl.BlockSpec((N_COMMANDS, D_MODEL), lambda i: (0, 0)),
            pl.BlockSpec((ARG_EMB_ROWS, 64), lambda i: (0, 0)),
            pl.BlockSpec((D_MODEL, 64 * N_ARGS), lambda i: (0, 0)),
            pl.BlockSpec((1, D_MODEL), lambda i: (0, 0)),
            pl.BlockSpec((GROUP_ROWS, D_MODEL), lambda i: (0, 0)),
            pl.BlockSpec((SB, D_MODEL), lambda i: (i, 0)),
        ],
        out_specs=pl.BlockSpec((NT, D_MODEL), lambda i: (i, 0)),
        out_shape=jax.ShapeDtypeStruct((T, D_MODEL), jnp.float32),
    )(cmdf, argsf, grpf, ce, ae, w, bf, ge, pe)
    return out.reshape(S, GN, D_MODEL)
